# baseline (speedup 1.0000x reference)
import jax
import jax.numpy as jnp
from jax import lax
from jax.experimental import pallas as pl
from jax.experimental.pallas import tpu as pltpu
from jax.experimental.pallas import tpu_sc as plsc

BATCH = 16384
F = 32
L = 1_000_000
LS = 1000

_NC = 2
_NS = 16
_NW = _NC * _NS
_BPW = BATCH // _NW

_CHUNK = 512
_RPW = 61
_ALIGNED = _RPW * _CHUNK * _NW
_TAIL0 = 999936
_NGRP = BATCH // 16

_SROWS = BATCH + 16


def _body_a(pc_idx_hbm, wp_t_hbm, tail_p_hbm, stage_hbm,
            idxv, jlist, blist, chunk, rb, dix, sem):
    wid = lax.axis_index("s") * _NC + lax.axis_index("c")
    lo = wid * (_RPW * _CHUNK)
    hi = jnp.where(wid == _NW - 1, jnp.int32(L), lo + _RPW * _CHUNK)

    pltpu.sync_copy(pc_idx_hbm, idxv)

    def scan_body(i, cursor):
        j16 = idxv[pl.ds(i * 16, 16)]
        b16 = lax.iota(jnp.int32, 16) + i * 16
        m = (j16 >= lo) & (j16 < hi)
        n = plsc.all_reduce_population_count(m)[0]
        plsc.store_compressed(jlist.at[pl.ds(cursor, 16)], j16, mask=m)
        plsc.store_compressed(blist.at[pl.ds(cursor, 16)], b16, mask=m)
        return cursor + n

    count = lax.fori_loop(0, _NGRP, scan_body, jnp.int32(0), unroll=4)
    jlist[pl.ds(count, 16)] = jnp.full((16,), -1, jnp.int32)
    ngroups = (count + 15) // 16

    nk = jnp.where(wid == _NW - 1, 63, _RPW)

    def chunk_body(k, _):
        is_main = k < _RPW + 1
        fs = jnp.where(k < _RPW, lo + k * _CHUNK, jnp.int32(_ALIGNED))

        @pl.when(is_main)
        def _():
            fsa = pl.multiple_of(fs, 128)
            pltpu.sync_copy(wp_t_hbm.at[:, pl.ds(fsa, _CHUNK)], chunk)

        @pl.when(jnp.logical_not(is_main))
        def _():
            pltpu.sync_copy(tail_p_hbm, chunk.at[:, pl.ds(0, 128)])

        lbase = jnp.where(is_main, fs, jnp.int32(_TAIL0))
        lhi = jnp.where(is_main, fs + _CHUNK, jnp.int32(L))

        def inner(g, _):
            j16 = jlist[pl.ds(g * 16, 16)]
            b16 = blist[pl.ds(g * 16, 16)]
            m = (j16 >= lbase) & (j16 < lhi)
            n = plsc.all_reduce_population_count(m)[0]

            @pl.when(n > 0)
            def _():
                l16 = jnp.where(m, j16 - lbase, 0)
                lane16 = lax.iota(jnp.int32, 16)
                for f in range(F):
                    vals = plsc.load_gather(
                        chunk, [jnp.full((16,), f, jnp.int32), l16])
                    plsc.store_scatter(
                        rb, [lane16, jnp.full((16,), f, jnp.int32)], vals)
                dix[0, :] = jnp.where(m, b16, _SROWS - 16 + lane16)
                pltpu.async_copy(rb, stage_hbm.at[dix.at[0]], sem).wait()

            return 0

        lax.fori_loop(0, ngroups, inner, 0)
        return 0

    lax.fori_loop(0, nk, chunk_body, 0)


def _body_b(stage_hbm, stars_idx_hbm, ws_t_hbm, out_hbm,
            sidx, sbuf, stvmem, outblock, sem):
    wid = lax.axis_index("s") * _NC + lax.axis_index("c")
    b0 = wid * _BPW

    pltpu.sync_copy(stars_idx_hbm.at[pl.ds(b0, _BPW)], sidx)
    pltpu.sync_copy(ws_t_hbm, sbuf)

    half = _BPW // 2
    for h in range(2):
        pltpu.sync_copy(stage_hbm.at[pl.ds(b0 + h * half, half)], stvmem)

        def transpose_group(g, _, h=h):
            gg = g + h * (half // 16)
            b16l = lax.iota(jnp.int32, 16) + g * 16
            j16 = sidx[pl.ds(gg * 16, 16)]
            for f in range(F):
                svals = plsc.load_gather(
                    sbuf, [jnp.full((16,), f, jnp.int32), j16])
                outblock[f, pl.ds(gg * 16, 16)] = svals
                pvals = plsc.load_gather(
                    stvmem, [b16l, jnp.full((16,), f, jnp.int32)])
                outblock[F + f, pl.ds(gg * 16, 16)] = pvals
            return 0

        lax.fori_loop(0, half // 16, transpose_group, 0)

    pltpu.sync_copy(outblock, out_hbm.at[:, pl.ds(b0, _BPW)])


@jax.jit
def _run(stars_idx, postalcode_idx, W_stars, W_postalcode):
    mesh = plsc.VectorSubcoreMesh(core_axis_name="c", subcore_axis_name="s")
    params = pltpu.CompilerParams(needs_layout_passes=False)

    ka = pl.kernel(
        _body_a,
        out_type=jax.ShapeDtypeStruct((_SROWS, 128), jnp.float32),
        mesh=mesh,
        scratch_types=[
            pltpu.VMEM((BATCH,), jnp.int32),
            pltpu.VMEM((BATCH + 16,), jnp.int32),
            pltpu.VMEM((BATCH + 16,), jnp.int32),
            pltpu.VMEM((F, _CHUNK), jnp.float32),
            pltpu.VMEM((16, 128), jnp.float32),
            pltpu.VMEM((1, 16), jnp.int32),
            pltpu.SemaphoreType.DMA,
        ],
        compiler_params=params,
    )
    kb = pl.kernel(
        _body_b,
        out_type=jax.ShapeDtypeStruct((2 * F, BATCH), jnp.float32),
        mesh=mesh,
        scratch_types=[
            pltpu.VMEM((_BPW,), jnp.int32),
            pltpu.VMEM((F, LS), jnp.float32),
            pltpu.VMEM((_BPW // 2, 128), jnp.float32),
            pltpu.VMEM((2 * F, _BPW), jnp.float32),
            pltpu.SemaphoreType.DMA,
        ],
        compiler_params=params,
    )

    pc_idx = postalcode_idx.astype(jnp.int32)
    s_idx = stars_idx.astype(jnp.int32)
    wp_t = W_postalcode.T
    ws_t = W_stars.T
    tail_p = jnp.pad(W_postalcode[_TAIL0:].T, ((0, 0), (0, 128 - (L - _TAIL0))))

    stage = ka(pc_idx, wp_t, tail_p)
    out_t = kb(stage, s_idx, ws_t)
    return out_t.T


def kernel(stars_idx, postalcode_idx, W_stars, W_postalcode):
    return _run(stars_idx, postalcode_idx, W_stars, W_postalcode)


# speedup vs baseline: 3.2683x; 3.2683x over previous
import jax
import jax.numpy as jnp
from jax import lax
from jax.experimental import pallas as pl
from jax.experimental.pallas import tpu as pltpu
from jax.experimental.pallas import tpu_sc as plsc

BATCH = 16384
F = 32
L = 1_000_000
LS = 1000

_NC = 2
_NS = 16
_NW = _NC * _NS
_BPW = BATCH // _NW

_CHUNK = 512
_RPW = 61
_TAIL0 = 999936
_NGRP = BATCH // 16
_RING = 8

_SROWS = BATCH + 16


def _body_a(pc_idx_hbm, wp_t_hbm, tail_p_hbm, stage_hbm,
            idxv, jlist, blist, cblist, chunkring, rbring, dixring,
            fsem, ssem):
    wid = lax.axis_index("s") * _NC + lax.axis_index("c")
    lo = wid * (_RPW * _CHUNK)
    hi = jnp.where(wid == _NW - 1, jnp.int32(L), lo + _RPW * _CHUNK)
    lane16 = lax.iota(jnp.int32, 16)

    pltpu.sync_copy(pc_idx_hbm, idxv.at[pl.ds(0, BATCH)])

    def scan_body(i, cursor):
        j16 = idxv[pl.ds(i * 16, 16)]
        b16 = lane16 + i * 16
        m = (j16 >= lo) & (j16 < hi)
        n = plsc.all_reduce_population_count(m)[0]
        plsc.store_compressed(jlist.at[pl.ds(cursor, 16)], j16, mask=m)
        plsc.store_compressed(blist.at[pl.ds(cursor, 16)], b16, mask=m)
        return cursor + n

    count = lax.fori_loop(0, _NGRP, scan_body, jnp.int32(0), unroll=4)
    jlist[pl.ds(count, 16)] = jnp.full((16,), -1, jnp.int32)
    ngroups = (count + 15) // 16

    nk = jnp.where(wid == _NW - 1, _RPW + 1, _RPW)

    def fetch(k):
        fs = pl.multiple_of(lo + k * _CHUNK, 128)
        pltpu.async_copy(wp_t_hbm.at[:, pl.ds(fs, _CHUNK)],
                         chunkring.at[lax.rem(k, 2)], fsem)

    def process_chunk(chunk, lbase, lhi, fired0):
        def comp(g, cur):
            j16 = jlist[pl.ds(g * 16, 16)]
            b16 = blist[pl.ds(g * 16, 16)]
            m = (j16 >= lbase) & (j16 < lhi)
            n = plsc.all_reduce_population_count(m)[0]
            plsc.store_compressed(idxv.at[pl.ds(cur, 16)], j16, mask=m)
            plsc.store_compressed(cblist.at[pl.ds(cur, 16)], b16, mask=m)
            return cur + n

        cnt = lax.fori_loop(0, ngroups, comp, jnp.int32(0))

        def ext(e, fired):
            j16 = idxv[pl.ds(e * 16, 16)]
            b16 = cblist[pl.ds(e * 16, 16)]
            m = lane16 < (cnt - e * 16)
            l16 = jnp.where(m, j16 - lbase, 0)
            slot = lax.rem(fired, _RING)

            @pl.when(fired >= _RING)
            def _():
                pltpu.make_async_copy(stage_hbm.at[pl.ds(0, 16)],
                                      rbring.at[0], ssem).wait()

            rb = rbring.at[slot]
            dix = dixring.at[slot]
            for f in range(F):
                vals = plsc.load_gather(
                    chunk, [jnp.full((16,), f, jnp.int32), l16])
                plsc.store_scatter(
                    rb, [lane16, jnp.full((16,), f, jnp.int32)], vals)
            dix[0, :] = jnp.where(m, b16, _SROWS - 16 + lane16)
            pltpu.async_copy(rb, stage_hbm.at[dix.at[0]], ssem)
            return fired + 1

        return lax.fori_loop(0, (cnt + 15) // 16, ext, fired0)

    fetch(0)

    def chunk_body(k, fired):
        @pl.when(k + 1 < nk)
        def _():
            fetch(k + 1)

        pltpu.make_async_copy(wp_t_hbm.at[:, pl.ds(0, _CHUNK)],
                              chunkring.at[0], fsem).wait()
        fs = lo + k * _CHUNK
        return process_chunk(chunkring.at[lax.rem(k, 2)], fs, fs + _CHUNK,
                             fired)

    fired = lax.fori_loop(0, nk, chunk_body, jnp.int32(0))

    def do_tail(f0):
        pltpu.sync_copy(tail_p_hbm, chunkring.at[0].at[:, pl.ds(0, 128)])
        return process_chunk(chunkring.at[0], jnp.int32(_TAIL0), jnp.int32(L),
                             f0)

    fired = lax.cond(wid == _NW - 1, do_tail, lambda f: f, fired)

    def drain(i, _):
        pltpu.make_async_copy(stage_hbm.at[pl.ds(0, 16)],
                              rbring.at[0], ssem).wait()
        return 0

    lax.fori_loop(0, jnp.minimum(fired, _RING), drain, 0)


def _body_b(stage_hbm, stars_idx_hbm, ws_t_hbm, out_hbm,
            sidx, sbuf, stvmem, outblock, sem):
    wid = lax.axis_index("s") * _NC + lax.axis_index("c")
    b0 = wid * _BPW
    half = _BPW // 2

    c1 = pltpu.async_copy(stars_idx_hbm.at[pl.ds(b0, _BPW)], sidx, sem)
    c2 = pltpu.async_copy(ws_t_hbm, sbuf, sem)
    c3 = pltpu.async_copy(stage_hbm.at[pl.ds(b0, half)], stvmem, sem)
    c1.wait()
    c2.wait()
    c3.wait()

    for h in range(2):
        if h:
            pltpu.sync_copy(stage_hbm.at[pl.ds(b0 + h * half, half)], stvmem)

        def transpose_group(g, _, h=h):
            gg = g + h * (half // 16)
            b16l = lax.iota(jnp.int32, 16) + g * 16
            j16 = sidx[pl.ds(gg * 16, 16)]
            for f in range(F):
                svals = plsc.load_gather(
                    sbuf, [jnp.full((16,), f, jnp.int32), j16])
                outblock[f, pl.ds(gg * 16, 16)] = svals
                pvals = plsc.load_gather(
                    stvmem, [b16l, jnp.full((16,), f, jnp.int32)])
                outblock[F + f, pl.ds(gg * 16, 16)] = pvals
            return 0

        lax.fori_loop(0, half // 16, transpose_group, 0)

    pltpu.sync_copy(outblock, out_hbm.at[:, pl.ds(b0, _BPW)])


@jax.jit
def _run(stars_idx, postalcode_idx, W_stars, W_postalcode):
    mesh = plsc.VectorSubcoreMesh(core_axis_name="c", subcore_axis_name="s")
    params = pltpu.CompilerParams(needs_layout_passes=False)

    ka = pl.kernel(
        _body_a,
        out_type=jax.ShapeDtypeStruct((_SROWS, 128), jnp.float32),
        mesh=mesh,
        scratch_types=[
            pltpu.VMEM((BATCH + 16,), jnp.int32),
            pltpu.VMEM((BATCH + 16,), jnp.int32),
            pltpu.VMEM((BATCH + 16,), jnp.int32),
            pltpu.VMEM((BATCH + 16,), jnp.int32),
            pltpu.VMEM((2, F, _CHUNK), jnp.float32),
            pltpu.VMEM((_RING, 16, 128), jnp.float32),
            pltpu.VMEM((_RING, 1, 16), jnp.int32),
            pltpu.SemaphoreType.DMA,
            pltpu.SemaphoreType.DMA,
        ],
        compiler_params=params,
    )
    kb = pl.kernel(
        _body_b,
        out_type=jax.ShapeDtypeStruct((2 * F, BATCH), jnp.float32),
        mesh=mesh,
        scratch_types=[
            pltpu.VMEM((_BPW,), jnp.int32),
            pltpu.VMEM((F, LS), jnp.float32),
            pltpu.VMEM((_BPW // 2, 128), jnp.float32),
            pltpu.VMEM((2 * F, _BPW), jnp.float32),
            pltpu.SemaphoreType.DMA,
        ],
        compiler_params=params,
    )

    pc_idx = postalcode_idx.astype(jnp.int32)
    s_idx = stars_idx.astype(jnp.int32)
    wp_t = W_postalcode.T
    ws_t = W_stars.T
    tail_p = jnp.pad(W_postalcode[_TAIL0:].T, ((0, 0), (0, 128 - (L - _TAIL0))))

    stage = ka(pc_idx, wp_t, tail_p)
    out_t = kb(stage, s_idx, ws_t)
    return out_t.T


def kernel(stars_idx, postalcode_idx, W_stars, W_postalcode):
    return _run(stars_idx, postalcode_idx, W_stars, W_postalcode)


# speedup vs baseline: 3.3324x; 1.0196x over previous
import jax
import jax.numpy as jnp
from jax import lax
from jax.experimental import pallas as pl
from jax.experimental.pallas import tpu as pltpu
from jax.experimental.pallas import tpu_sc as plsc

BATCH = 16384
F = 32
L = 1_000_000
LS = 1000

_NC = 2
_NS = 16
_NW = _NC * _NS
_BPW = BATCH // _NW

_CHUNK = 512
_RPW = 61
_TAIL0 = 999936
_NGRP = BATCH // 16
_CRING = 4
_RING = 8
_BSH = 14

_SROWS = BATCH + 16


def _body_a(pc_idx_hbm, wp_t_hbm, tail_p_hbm, stage_hbm,
            idxv, plist, chunkring, rbring, dixring, fsem, ssem):
    wid = lax.axis_index("s") * _NC + lax.axis_index("c")
    lo = wid * (_RPW * _CHUNK)
    hi = jnp.where(wid == _NW - 1, jnp.int32(L), lo + _RPW * _CHUNK)
    lane16 = lax.iota(jnp.int32, 16)

    pltpu.sync_copy(pc_idx_hbm, idxv.at[pl.ds(0, BATCH)])

    def scan_body(i, cursor):
        j16 = idxv[pl.ds(i * 16, 16)]
        b16 = lane16 + i * 16
        m = (j16 >= lo) & (j16 < hi)
        n = plsc.all_reduce_population_count(m)[0]
        p16 = ((j16 - lo) << _BSH) | b16
        plsc.store_compressed(plist.at[pl.ds(cursor, 16)], p16, mask=m)
        return cursor + n

    count = lax.fori_loop(0, _NGRP, scan_body, jnp.int32(0), unroll=4)
    plist[pl.ds(count, 16)] = jnp.full((16,), -1, jnp.int32)
    ngroups = (count + 15) // 16

    nk = jnp.where(wid == _NW - 1, _RPW + 1, _RPW)

    def fetch(k):
        fs = pl.multiple_of(lo + k * _CHUNK, 128)
        pltpu.async_copy(wp_t_hbm.at[:, pl.ds(fs, _CHUNK)],
                         chunkring.at[lax.rem(k, _CRING)], fsem)

    def process_chunk(chunk, rlo, rhi, fired0):
        plo = rlo << _BSH
        phi = rhi << _BSH

        def comp(g, cur):
            p16 = plist[pl.ds(g * 16, 16)]
            m = (p16 >= plo) & (p16 < phi)
            n = plsc.all_reduce_population_count(m)[0]
            plsc.store_compressed(idxv.at[pl.ds(cur, 16)], p16, mask=m)
            return cur + n

        cnt = lax.fori_loop(0, ngroups, comp, jnp.int32(0))

        def ext(e, fired):
            p16 = idxv[pl.ds(e * 16, 16)]
            m = lane16 < (cnt - e * 16)
            l16 = jnp.where(m, (p16 >> _BSH) - rlo, 0)
            b16 = p16 & ((1 << _BSH) - 1)
            slot = lax.rem(fired, _RING)

            @pl.when(fired >= _RING)
            def _():
                pltpu.make_async_copy(stage_hbm.at[pl.ds(0, 16)],
                                      rbring.at[0], ssem).wait()

            rb = rbring.at[slot]
            dix = dixring.at[slot]
            for f in range(F):
                vals = plsc.load_gather(
                    chunk, [jnp.full((16,), f, jnp.int32), l16])
                plsc.store_scatter(
                    rb, [lane16, jnp.full((16,), f, jnp.int32)], vals)
            dix[0, :] = jnp.where(m, b16, _SROWS - 16 + lane16)
            pltpu.async_copy(rb, stage_hbm.at[dix.at[0]], ssem)
            return fired + 1

        return lax.fori_loop(0, (cnt + 15) // 16, ext, fired0)

    for k0 in range(_CRING - 1):
        @pl.when(k0 < nk)
        def _(k0=k0):
            fetch(k0)

    def chunk_body(k, fired):
        @pl.when(k + (_CRING - 1) < nk)
        def _():
            fetch(k + (_CRING - 1))

        pltpu.make_async_copy(wp_t_hbm.at[:, pl.ds(0, _CHUNK)],
                              chunkring.at[0], fsem).wait()
        rlo = k * _CHUNK
        return process_chunk(chunkring.at[lax.rem(k, _CRING)], rlo,
                             rlo + _CHUNK, fired)

    fired = lax.fori_loop(0, nk, chunk_body, jnp.int32(0))

    def do_tail(f0):
        pltpu.sync_copy(tail_p_hbm, chunkring.at[0].at[:, pl.ds(0, 128)])
        return process_chunk(chunkring.at[0], jnp.int32(_TAIL0 - 968192),
                             jnp.int32(L - 968192), f0)

    fired = lax.cond(wid == _NW - 1, do_tail, lambda f: f, fired)

    def drain(i, _):
        pltpu.make_async_copy(stage_hbm.at[pl.ds(0, 16)],
                              rbring.at[0], ssem).wait()
        return 0

    lax.fori_loop(0, jnp.minimum(fired, _RING), drain, 0)


def _body_b(stage_hbm, stars_idx_hbm, ws_t_hbm, out_hbm,
            sidx, sbuf, stvring, outblock, fsem, sem):
    wid = lax.axis_index("s") * _NC + lax.axis_index("c")
    b0 = wid * _BPW
    quarter = _BPW // 4
    nq = 4

    c1 = pltpu.async_copy(stars_idx_hbm.at[pl.ds(b0, _BPW)], sidx, sem)
    c2 = pltpu.async_copy(ws_t_hbm, sbuf, sem)
    pltpu.async_copy(stage_hbm.at[pl.ds(b0, quarter)], stvring.at[0], fsem)
    c1.wait()
    c2.wait()

    for q in range(nq):
        if q + 1 < nq:
            pltpu.async_copy(
                stage_hbm.at[pl.ds(b0 + (q + 1) * quarter, quarter)],
                stvring.at[(q + 1) % 2], fsem)
        pltpu.make_async_copy(stage_hbm.at[pl.ds(0, quarter)],
                              stvring.at[0], fsem).wait()
        stv = stvring.at[q % 2]

        def transpose_group(g, _, q=q, stv=stv):
            gg = g + q * (quarter // 16)
            b16l = lax.iota(jnp.int32, 16) + g * 16
            j16 = sidx[pl.ds(gg * 16, 16)]
            for f in range(F):
                svals = plsc.load_gather(
                    sbuf, [jnp.full((16,), f, jnp.int32), j16])
                outblock[f, pl.ds(gg * 16, 16)] = svals
                pvals = plsc.load_gather(
                    stv, [b16l, jnp.full((16,), f, jnp.int32)])
                outblock[F + f, pl.ds(gg * 16, 16)] = pvals
            return 0

        lax.fori_loop(0, quarter // 16, transpose_group, 0)

    pltpu.sync_copy(outblock, out_hbm.at[:, pl.ds(b0, _BPW)])


@jax.jit
def _run(stars_idx, postalcode_idx, W_stars, W_postalcode):
    mesh = plsc.VectorSubcoreMesh(core_axis_name="c", subcore_axis_name="s")
    params = pltpu.CompilerParams(needs_layout_passes=False)

    ka = pl.kernel(
        _body_a,
        out_type=jax.ShapeDtypeStruct((_SROWS, 128), jnp.float32),
        mesh=mesh,
        scratch_types=[
            pltpu.VMEM((BATCH + 16,), jnp.int32),
            pltpu.VMEM((BATCH + 16,), jnp.int32),
            pltpu.VMEM((_CRING, F, _CHUNK), jnp.float32),
            pltpu.VMEM((_RING, 16, 128), jnp.float32),
            pltpu.VMEM((_RING, 1, 16), jnp.int32),
            pltpu.SemaphoreType.DMA,
            pltpu.SemaphoreType.DMA,
        ],
        compiler_params=params,
    )
    kb = pl.kernel(
        _body_b,
        out_type=jax.ShapeDtypeStruct((2 * F, BATCH), jnp.float32),
        mesh=mesh,
        scratch_types=[
            pltpu.VMEM((_BPW,), jnp.int32),
            pltpu.VMEM((F, LS), jnp.float32),
            pltpu.VMEM((2, _BPW // 4, 128), jnp.float32),
            pltpu.VMEM((2 * F, _BPW), jnp.float32),
            pltpu.SemaphoreType.DMA,
            pltpu.SemaphoreType.DMA,
        ],
        compiler_params=params,
    )

    pc_idx = postalcode_idx.astype(jnp.int32)
    s_idx = stars_idx.astype(jnp.int32)
    wp_t = W_postalcode.T
    ws_t = W_stars.T
    tail_p = jnp.pad(W_postalcode[_TAIL0:].T, ((0, 0), (0, 128 - (L - _TAIL0))))

    stage = ka(pc_idx, wp_t, tail_p)
    out_t = kb(stage, s_idx, ws_t)
    return out_t.T


def kernel(stars_idx, postalcode_idx, W_stars, W_postalcode):
    return _run(stars_idx, postalcode_idx, W_stars, W_postalcode)


# speedup vs baseline: 4.5302x; 1.3594x over previous
import jax
import jax.numpy as jnp
from jax import lax
from jax.experimental import pallas as pl
from jax.experimental.pallas import tpu as pltpu
from jax.experimental.pallas import tpu_sc as plsc

BATCH = 16384
F = 32
L = 1_000_000
LS = 1000

_NC = 2
_NS = 16
_NW = _NC * _NS
_BPW = BATCH // _NW

_CHUNK = 1024
_RPW = 30
_LPW = 31232
_TAIL0 = 999936
_NGRP = BATCH // 16
_CRING = 2
_RING = 6
_BSH = 14

_SROWS = BATCH + 16


def _body_a(pc_idx_hbm, wp_t_hbm, tail_p_hbm, stage_hbm,
            idxv, plist, chunkring, epi, rbring, dixring, fsem, esem, ssem):
    wid = lax.axis_index("s") * _NC + lax.axis_index("c")
    lo = wid * _LPW
    hi = jnp.where(wid == _NW - 1, jnp.int32(L), lo + _LPW)
    lane16 = lax.iota(jnp.int32, 16)

    def fetch(k):
        fs = pl.multiple_of(lo + k * _CHUNK, 128)
        pltpu.async_copy(wp_t_hbm.at[:, pl.ds(fs, _CHUNK)],
                         chunkring.at[lax.rem(k, _CRING)], fsem)

    fetch(0)
    pltpu.async_copy(
        wp_t_hbm.at[:, pl.ds(pl.multiple_of(lo + _RPW * _CHUNK, 128), 512)],
        epi, esem)

    pltpu.sync_copy(pc_idx_hbm, idxv.at[pl.ds(0, BATCH)])

    def scan_body(i, cursor):
        j16 = idxv[pl.ds(i * 16, 16)]
        b16 = lane16 + i * 16
        m = (j16 >= lo) & (j16 < hi)
        n = plsc.all_reduce_population_count(m)[0]
        p16 = ((j16 - lo) << _BSH) | b16
        plsc.store_compressed(plist.at[pl.ds(cursor, 16)], p16, mask=m)
        return cursor + n

    count = lax.fori_loop(0, _NGRP, scan_body, jnp.int32(0), unroll=4)
    plist[pl.ds(count, 16)] = jnp.full((16,), -1, jnp.int32)
    ngroups = (count + 15) // 16

    def process_chunk(chunk, rlo, rhi, fired0, lmax=_CHUNK - 1):
        plo = rlo << _BSH
        phi = rhi << _BSH

        def comp(g, cur):
            p16 = plist[pl.ds(g * 16, 16)]
            m = (p16 >= plo) & (p16 < phi)
            n = plsc.all_reduce_population_count(m)[0]
            plsc.store_compressed(idxv.at[pl.ds(cur, 16)], p16, mask=m)
            return cur + n

        cnt = lax.fori_loop(0, ngroups, comp, jnp.int32(0))

        def ext(e, fired):
            base = e * 16
            p16 = idxv[pl.ds(base, 16)]
            m = lane16 < (cnt - base)
            b16 = p16 & ((1 << _BSH) - 1)
            slot = lax.rem(fired, _RING)

            @pl.when(fired >= _RING)
            def _():
                pltpu.make_async_copy(stage_hbm.at[pl.ds(0, 16)],
                                      rbring.at[0], ssem).wait()

            rb = rbring.at[slot]
            dix = dixring.at[slot]
            for ee in range(16):
                p = p16[ee]
                l = lax.max(jnp.int32(0),
                            lax.min((p >> _BSH) - rlo, jnp.int32(lmax)))
                lb = jnp.broadcast_to(l, (16,))
                rb[ee, pl.ds(0, 16)] = plsc.load_gather(chunk, [lane16, lb])
                rb[ee, pl.ds(16, 16)] = plsc.load_gather(
                    chunk, [lane16 + 16, lb])
            dix[0, :] = jnp.where(m, b16, _SROWS - 16 + lane16)
            pltpu.async_copy(rb, stage_hbm.at[dix.at[0]], ssem)
            return fired + 1

        return lax.fori_loop(0, (cnt + 15) // 16, ext, fired0)

    def chunk_body(k, fired):
        @pl.when(k + (_CRING - 1) < _RPW)
        def _():
            fetch(k + (_CRING - 1))

        pltpu.make_async_copy(wp_t_hbm.at[:, pl.ds(0, _CHUNK)],
                              chunkring.at[0], fsem).wait()
        rlo = k * _CHUNK
        return process_chunk(chunkring.at[lax.rem(k, _CRING)], rlo,
                             rlo + _CHUNK, fired)

    fired = lax.fori_loop(0, _RPW, chunk_body, jnp.int32(0))

    pltpu.make_async_copy(wp_t_hbm.at[:, pl.ds(0, 512)], epi, esem).wait()
    fired = process_chunk(epi, jnp.int32(_RPW * _CHUNK),
                          jnp.int32(_LPW), fired, lmax=511)

    def do_extra(f0):
        pltpu.sync_copy(wp_t_hbm.at[:, pl.ds(999424, 512)], epi)
        f1 = process_chunk(epi, jnp.int32(_LPW), jnp.int32(_LPW + 512), f0,
                           lmax=511)
        pltpu.sync_copy(tail_p_hbm, epi.at[:, pl.ds(0, 128)])
        return process_chunk(epi, jnp.int32(_TAIL0 - 968192),
                             jnp.int32(L - 968192), f1, lmax=127)

    fired = lax.cond(wid == _NW - 1, do_extra, lambda f: f, fired)

    def drain(i, _):
        pltpu.make_async_copy(stage_hbm.at[pl.ds(0, 16)],
                              rbring.at[0], ssem).wait()
        return 0

    lax.fori_loop(0, jnp.minimum(fired, _RING), drain, 0)


def _body_b(stage_hbm, stars_idx_hbm, ws_t_hbm, out_hbm,
            sidx, sbuf, stvring, outblock, fsem, sem):
    wid = lax.axis_index("s") * _NC + lax.axis_index("c")
    b0 = wid * _BPW
    quarter = _BPW // 4
    nq = 4

    c1 = pltpu.async_copy(stars_idx_hbm.at[pl.ds(b0, _BPW)], sidx, sem)
    c2 = pltpu.async_copy(ws_t_hbm, sbuf, sem)
    pltpu.async_copy(stage_hbm.at[pl.ds(b0, quarter)], stvring.at[0], fsem)
    c1.wait()
    c2.wait()

    for q in range(nq):
        if q + 1 < nq:
            pltpu.async_copy(
                stage_hbm.at[pl.ds(b0 + (q + 1) * quarter, quarter)],
                stvring.at[(q + 1) % 2], fsem)
        pltpu.make_async_copy(stage_hbm.at[pl.ds(0, quarter)],
                              stvring.at[0], fsem).wait()
        stv = stvring.at[q % 2]

        def transpose_group(g, _, q=q, stv=stv):
            gg = g + q * (quarter // 16)
            b16l = lax.iota(jnp.int32, 16) + g * 16
            j16 = sidx[pl.ds(gg * 16, 16)]
            for f in range(F):
                svals = plsc.load_gather(
                    sbuf, [jnp.full((16,), f, jnp.int32), j16])
                outblock[f, pl.ds(gg * 16, 16)] = svals
                pvals = plsc.load_gather(
                    stv, [b16l, jnp.full((16,), f, jnp.int32)])
                outblock[F + f, pl.ds(gg * 16, 16)] = pvals
            return 0

        lax.fori_loop(0, quarter // 16, transpose_group, 0)

    pltpu.sync_copy(outblock, out_hbm.at[:, pl.ds(b0, _BPW)])


@jax.jit
def _run(stars_idx, postalcode_idx, W_stars, W_postalcode):
    mesh = plsc.VectorSubcoreMesh(core_axis_name="c", subcore_axis_name="s")
    params = pltpu.CompilerParams(needs_layout_passes=False)

    ka = pl.kernel(
        _body_a,
        out_type=jax.ShapeDtypeStruct((_SROWS, 128), jnp.float32),
        mesh=mesh,
        scratch_types=[
            pltpu.VMEM((BATCH + 16,), jnp.int32),
            pltpu.VMEM((BATCH + 16,), jnp.int32),
            pltpu.VMEM((_CRING, F, _CHUNK), jnp.float32),
            pltpu.VMEM((F, 512), jnp.float32),
            pltpu.VMEM((_RING, 16, 128), jnp.float32),
            pltpu.VMEM((_RING, 1, 16), jnp.int32),
            pltpu.SemaphoreType.DMA,
            pltpu.SemaphoreType.DMA,
            pltpu.SemaphoreType.DMA,
        ],
        compiler_params=params,
    )
    kb = pl.kernel(
        _body_b,
        out_type=jax.ShapeDtypeStruct((2 * F, BATCH), jnp.float32),
        mesh=mesh,
        scratch_types=[
            pltpu.VMEM((_BPW,), jnp.int32),
            pltpu.VMEM((F, LS), jnp.float32),
            pltpu.VMEM((2, _BPW // 4, 128), jnp.float32),
            pltpu.VMEM((2 * F, _BPW), jnp.float32),
            pltpu.SemaphoreType.DMA,
            pltpu.SemaphoreType.DMA,
        ],
        compiler_params=params,
    )

    pc_idx = postalcode_idx.astype(jnp.int32)
    s_idx = stars_idx.astype(jnp.int32)
    wp_t = W_postalcode.T
    ws_t = W_stars.T
    tail_p = jnp.pad(W_postalcode[_TAIL0:].T, ((0, 0), (0, 128 - (L - _TAIL0))))

    stage = ka(pc_idx, wp_t, tail_p)
    out_t = kb(stage, s_idx, ws_t)
    return out_t.T


def kernel(stars_idx, postalcode_idx, W_stars, W_postalcode):
    return _run(stars_idx, postalcode_idx, W_stars, W_postalcode)
